# baseline (speedup 1.0000x reference)
import functools

import jax
import jax.numpy as jnp
from jax import lax
from jax.experimental import pallas as pl
from jax.experimental.pallas import tpu as pltpu
from jax.experimental.pallas import tpu_sc as plsc

FIELD = 26
EMB = 16
BATCH = 16384
VOCAB = 26 * 38462
BF = BATCH * FIELD
NW = 32
ROWS_PER_W = BF // (NW * 128)
SUB = 13
NGRP = ROWS_PER_W // SUB
EPS = 1e-5
FBLK = 16384
NSUP = (VOCAB + FBLK - 1) // FBLK
VP = NSUP * FBLK


def _tc_idx(xT):
    def body(x_ref, out_ref):
        for f in range(FIELD):
            out_ref[pl.ds(f * (BATCH // 128), BATCH // 128), :] = (
                x_ref[f:f + 1, :].reshape(BATCH // 128, 128))

    return pl.pallas_call(
        body,
        grid=(1,),
        in_specs=[pl.BlockSpec((FIELD, BATCH), lambda f: (0, 0))],
        out_specs=pl.BlockSpec((BF // 128, 128), lambda f: (0, 0)),
        out_shape=jax.ShapeDtypeStruct((BF // 128, 128), jnp.int32),
    )(xT)


def _tc_format2(tabT):
    def body(t_ref, out2_ref):
        eye = jax.lax.broadcasted_iota(jnp.int32, (128, 128), 0)
        eye = jnp.where(eye == jax.lax.broadcasted_iota(
            jnp.int32, (128, 128), 1), 1.0, 0.0).astype(jnp.float32)
        for w in range(FBLK // 1024):
            s = jnp.concatenate(
                [t_ref[:, w * 1024 + m * 128: w * 1024 + (m + 1) * 128]
                 for m in range(8)], axis=0)
            r = jax.lax.dot_general(
                s, eye, (((0,), (0,)), ((), ())),
                preferred_element_type=jnp.float32)
            out2_ref[pl.ds(w * 128, 128), :] = r

    return pl.pallas_call(
        body,
        grid=(NSUP,),
        in_specs=[pl.BlockSpec((EMB, FBLK), lambda j: (0, j))],
        out_specs=pl.BlockSpec((FBLK * EMB // 128, 128), lambda j: (j, 0)),
        out_shape=jax.ShapeDtypeStruct((VP * EMB // 128, 128), jnp.float32),
    )(tabT)


def _sc_first(idx2d, tab1):
    mesh = plsc.VectorSubcoreMesh(core_axis_name="c", subcore_axis_name="s")

    @functools.partial(
        pl.kernel,
        out_type=[
            jax.ShapeDtypeStruct((BF // 128, 128), jnp.int32),
            jax.ShapeDtypeStruct((BF // 128, 128, 1), jnp.float32),
        ],
        mesh=mesh,
        compiler_params=pltpu.CompilerParams(use_tc_tiling_on_sc=False),
        scratch_types=[
            pltpu.VMEM((ROWS_PER_W, 128), jnp.int32),
            pltpu.VMEM((ROWS_PER_W, 128), jnp.int32),
            pltpu.VMEM((SUB, 128, 1), jnp.float32),
            pltpu.SemaphoreType.DMA,
        ],
    )
    def k(idx_hbm, tab1_hbm, pidx_out, first_out, idx_v, pidx_v, fv, sem):
        wid = lax.axis_index("s") * 2 + lax.axis_index("c")
        row0 = wid * ROWS_PER_W
        pltpu.sync_copy(idx_hbm.at[pl.ds(row0, ROWS_PER_W)], idx_v)

        def xform(i, carry):
            for s in range(8):
                v = idx_v[i, pl.ds(s * 16, 16)]
                p = (v & -1024) + ((v & 127) << 3) + ((v >> 7) & 7)
                pidx_v[i, pl.ds(s * 16, 16)] = p
            return carry

        lax.fori_loop(0, ROWS_PER_W, xform, 0)
        pltpu.sync_copy(pidx_v, pidx_out.at[pl.ds(row0, ROWS_PER_W)])

        def first_grp(g, carry):
            handles = []
            for b in range(SUB):
                handles.append(pltpu.async_copy(
                    tab1_hbm.at[idx_v.at[g * SUB + b]], fv.at[b], sem))
            for h in handles:
                h.wait()
            pltpu.sync_copy(fv, first_out.at[pl.ds(row0 + g * SUB, SUB)])
            return carry

        lax.fori_loop(0, NGRP, first_grp, 0)

    return k(idx2d, tab1)


def _sc_emb(pidx2d, tab2):
    mesh = plsc.VectorSubcoreMesh(core_axis_name="c", subcore_axis_name="s")

    @functools.partial(
        pl.kernel,
        out_type=jax.ShapeDtypeStruct((BF // 128, 128, EMB), jnp.float32),
        mesh=mesh,
        compiler_params=pltpu.CompilerParams(use_tc_tiling_on_sc=False),
        scratch_types=[
            pltpu.VMEM((ROWS_PER_W, 128), jnp.int32),
            pltpu.VMEM((SUB, 128, EMB), jnp.float32),
            pltpu.SemaphoreType.DMA,
        ],
    )
    def k(pidx_hbm, tab2_hbm, emb_out, pidx_v, rows_v, sem):
        wid = lax.axis_index("s") * 2 + lax.axis_index("c")
        row0 = wid * ROWS_PER_W
        pltpu.sync_copy(pidx_hbm.at[pl.ds(row0, ROWS_PER_W)], pidx_v)

        def emb_grp(g, carry):
            handles = []
            for b in range(SUB):
                handles.append(pltpu.async_copy(
                    tab2_hbm.at[pidx_v.at[g * SUB + b]], rows_v.at[b], sem))
            for h in handles:
                h.wait()
            pltpu.sync_copy(rows_v, emb_out.at[pl.ds(row0 + g * SUB, SUB)])
            return carry

        lax.fori_loop(0, NGRP, emb_grp, 0)

    return k(pidx2d, tab2)


def _tc_dense(emb3d, first3d, w1k, b1d, w2d, b2d, d16, d8):
    blk = 1024
    grid = BATCH // blk
    rows = blk * EMB // 128

    def body(e_ref, f_ref, w1_ref, b1_ref, w2_ref, b2_ref, d16_ref, d8_ref,
             fd_ref, fs_ref):
        e = e_ref[...]
        s3 = jnp.sum(e, axis=0)
        sq3 = jnp.sum(e * e, axis=0)
        d16 = d16_ref[...]
        fm2 = 0.5 * (jnp.dot(s3 * s3, d16, preferred_element_type=jnp.float32)
                     - jnp.dot(sq3, d16, preferred_element_type=jnp.float32))
        h1 = jnp.dot(e[0], w1_ref[0], preferred_element_type=jnp.float32)
        for f in range(1, FIELD):
            h1 = h1 + jnp.dot(e[f], w1_ref[f],
                              preferred_element_type=jnp.float32)
        h1 = jnp.maximum(h1 + b1_ref[...], 0.0)
        h2 = jnp.dot(h1, w2_ref[...], preferred_element_type=jnp.float32)
        h2 = jnp.maximum(h2 + b2_ref[...], 0.0)
        deep = jnp.dot(h2, d8_ref[...], preferred_element_type=jnp.float32)
        fd_ref[...] = fm2 + deep
        fs_ref[...] = jnp.sum(f_ref[...], axis=0)

    return pl.pallas_call(
        body,
        grid=(grid,),
        in_specs=[
            pl.BlockSpec((FIELD, rows, 128), lambda i: (0, i, 0)),
            pl.BlockSpec((FIELD, blk // 128, 128), lambda i: (0, i, 0)),
            pl.BlockSpec((FIELD, 128, 96), lambda i: (0, 0, 0)),
            pl.BlockSpec((96,), lambda i: (0,)),
            pl.BlockSpec((96, 64), lambda i: (0, 0)),
            pl.BlockSpec((64,), lambda i: (0,)),
            pl.BlockSpec((128, 8), lambda i: (0, 0)),
            pl.BlockSpec((64, 8), lambda i: (0, 0)),
        ],
        out_specs=[
            pl.BlockSpec((rows, 8), lambda i: (i, 0)),
            pl.BlockSpec((blk // 128, 128), lambda i: (i, 0)),
        ],
        out_shape=[
            jax.ShapeDtypeStruct((BATCH // 8, 8), jnp.float32),
            jax.ShapeDtypeStruct((BATCH // 128, 128), jnp.float32),
        ],
    )(emb3d, first3d, w1k, b1d, w2d, b2d, d16, d8)


def kernel(x, fm_first_w, fm_second_w, lin1_w, lin1_b, bn1_g, bn1_b,
           lin2_w, lin2_b, bn2_g, bn2_b, bias):
    idx2d = _tc_idx(x.T)
    pidx2d, first2 = _sc_first(idx2d, fm_first_w)
    out2 = _tc_format2(fm_second_w.T)
    emb3 = _sc_emb(pidx2d, out2.reshape(VP, EMB))
    emb3d = emb3.reshape(FIELD, BATCH * EMB // 128, 128)
    first3d = first2.reshape(FIELD, BATCH // 128, 128)

    a1 = bn1_g / jnp.sqrt(1.0 + EPS)
    w1f = (lin1_w * a1[None, :]).reshape(FIELD, EMB, 12)
    b1f = lin1_b * a1 + bn1_b
    a2 = bn2_g / jnp.sqrt(1.0 + EPS)
    w2f = lin2_w * a2[None, :]
    b2f = lin2_b * a2 + bn2_b
    eye8 = jnp.eye(8, dtype=jnp.float32)
    w1k = jnp.einsum('ab,fej->faebj', eye8, w1f).reshape(FIELD, 128, 96)
    b1d = jnp.tile(b1f, 8)
    w2d = jnp.einsum('ab,ej->aebj', eye8, w2f).reshape(96, 64)
    b2d = jnp.tile(b2f, 8)
    d16 = jnp.einsum('ab,e->aeb', eye8, jnp.ones(EMB)).reshape(128, 8)
    d8 = jnp.einsum('ab,e->aeb', eye8, jnp.ones(8)).reshape(64, 8)

    fd, fs = _tc_dense(emb3d, first3d, w1k, b1d, w2d, b2d, d16, d8)
    tot = fd.reshape(BATCH) + fs.reshape(BATCH) + bias[0]
    return jax.nn.sigmoid(tot)


# speedup vs baseline: 6.5883x; 6.5883x over previous
import functools

import jax
import jax.numpy as jnp
from jax import lax
from jax.experimental import pallas as pl
from jax.experimental.pallas import tpu as pltpu
from jax.experimental.pallas import tpu_sc as plsc

FIELD = 26
EMB = 16
BATCH = 16384
VOCAB = 26 * 38462
BF = BATCH * FIELD
NW = 32
ROWS_PER_W = BF // (NW * 128)
SUB = 13
NGRP = ROWS_PER_W // SUB
EPS = 1e-5
FBLK = 16384
NSUP = (VOCAB + FBLK - 1) // FBLK
VP = NSUP * FBLK


def _tc_idx(xT):
    def body(x_ref, out_ref):
        for f in range(FIELD):
            out_ref[pl.ds(f * (BATCH // 128), BATCH // 128), :] = (
                x_ref[f:f + 1, :].reshape(BATCH // 128, 128))

    return pl.pallas_call(
        body,
        grid=(1,),
        in_specs=[pl.BlockSpec((FIELD, BATCH), lambda f: (0, 0))],
        out_specs=pl.BlockSpec((BF // 128, 128), lambda f: (0, 0)),
        out_shape=jax.ShapeDtypeStruct((BF // 128, 128), jnp.int32),
    )(xT)


def _tc_format1(firstT):
    def body(f_ref, out1_ref):
        out1_ref[...] = f_ref[...].reshape(FBLK // 128, 128)

    return pl.pallas_call(
        body,
        grid=(NSUP,),
        in_specs=[pl.BlockSpec((1, FBLK), lambda j: (0, j))],
        out_specs=pl.BlockSpec((FBLK // 128, 128), lambda j: (j, 0)),
        out_shape=jax.ShapeDtypeStruct((VP // 128, 128), jnp.float32),
    )(firstT)


def _tc_format2(tabT):
    def body(t_ref, out2_ref):
        eye = jax.lax.broadcasted_iota(jnp.int32, (128, 128), 0)
        eye = jnp.where(eye == jax.lax.broadcasted_iota(
            jnp.int32, (128, 128), 1), 1.0, 0.0).astype(jnp.float32)
        for w in range(FBLK // 1024):
            s = jnp.concatenate(
                [t_ref[:, w * 1024 + m * 128: w * 1024 + (m + 1) * 128]
                 for m in range(8)], axis=0)
            r = jax.lax.dot_general(
                s, eye, (((0,), (0,)), ((), ())),
                preferred_element_type=jnp.float32)
            out2_ref[pl.ds(w * 128, 128), :] = r

    return pl.pallas_call(
        body,
        grid=(NSUP,),
        in_specs=[pl.BlockSpec((EMB, FBLK), lambda j: (0, j))],
        out_specs=pl.BlockSpec((FBLK * EMB // 128, 128), lambda j: (j, 0)),
        out_shape=jax.ShapeDtypeStruct((VP * EMB // 128, 128), jnp.float32),
    )(tabT)


def _sc_first(idx2d, tab1):
    mesh = plsc.VectorSubcoreMesh(core_axis_name="c", subcore_axis_name="s")

    @functools.partial(
        pl.kernel,
        out_type=[
            jax.ShapeDtypeStruct((BF // 128, 128), jnp.int32),
            jax.ShapeDtypeStruct((BF // 128, 128), jnp.float32),
        ],
        mesh=mesh,
        compiler_params=pltpu.CompilerParams(use_tc_tiling_on_sc=False),
        scratch_types=[
            pltpu.VMEM((ROWS_PER_W, 128), jnp.int32),
            pltpu.VMEM((ROWS_PER_W, 128), jnp.int32),
            pltpu.VMEM((ROWS_PER_W, 128), jnp.float32),
            pltpu.SemaphoreType.DMA,
        ],
    )
    def k(idx_hbm, tab1_hbm, pidx_out, first_out, idx_v, pidx_v, fv, sem):
        wid = lax.axis_index("s") * 2 + lax.axis_index("c")
        row0 = wid * ROWS_PER_W
        pltpu.sync_copy(idx_hbm.at[pl.ds(row0, ROWS_PER_W)], idx_v)

        def xform(i, carry):
            for s in range(8):
                v = idx_v[i, pl.ds(s * 16, 16)]
                p = (v & -1024) + ((v & 127) << 3) + ((v >> 7) & 7)
                pidx_v[i, pl.ds(s * 16, 16)] = p
            return carry

        lax.fori_loop(0, ROWS_PER_W, xform, 0)
        pltpu.sync_copy(pidx_v, pidx_out.at[pl.ds(row0, ROWS_PER_W)])

        def first_grp(g, carry):
            handles = []
            for b in range(SUB):
                handles.append(pltpu.async_copy(
                    tab1_hbm.at[idx_v.at[g * SUB + b]],
                    fv.at[g * SUB + b], sem))
            for h in handles:
                h.wait()
            return carry

        lax.fori_loop(0, NGRP, first_grp, 0)
        pltpu.sync_copy(fv, first_out.at[pl.ds(row0, ROWS_PER_W)])

    return k(idx2d, tab1)


def _sc_emb(pidx2d, tab2):
    mesh = plsc.VectorSubcoreMesh(core_axis_name="c", subcore_axis_name="s")

    @functools.partial(
        pl.kernel,
        out_type=jax.ShapeDtypeStruct((BF // 128, 128, EMB), jnp.float32),
        mesh=mesh,
        compiler_params=pltpu.CompilerParams(use_tc_tiling_on_sc=False),
        scratch_types=[
            pltpu.VMEM((ROWS_PER_W, 128), jnp.int32),
            pltpu.VMEM((SUB, 128, EMB), jnp.float32),
            pltpu.SemaphoreType.DMA,
        ],
    )
    def k(pidx_hbm, tab2_hbm, emb_out, pidx_v, rows_v, sem):
        wid = lax.axis_index("s") * 2 + lax.axis_index("c")
        row0 = wid * ROWS_PER_W
        pltpu.sync_copy(pidx_hbm.at[pl.ds(row0, ROWS_PER_W)], pidx_v)

        def emb_grp(g, carry):
            handles = []
            for b in range(SUB):
                handles.append(pltpu.async_copy(
                    tab2_hbm.at[pidx_v.at[g * SUB + b]], rows_v.at[b], sem))
            for h in handles:
                h.wait()
            pltpu.sync_copy(rows_v, emb_out.at[pl.ds(row0 + g * SUB, SUB)])
            return carry

        lax.fori_loop(0, NGRP, emb_grp, 0)

    return k(pidx2d, tab2)


def _tc_dense(emb3d, first3d, w1k, b1d, w2d, b2d, d16, d8):
    blk = 1024
    grid = BATCH // blk
    rows = blk * EMB // 128

    def body(e_ref, f_ref, w1_ref, b1_ref, w2_ref, b2_ref, d16_ref, d8_ref,
             fd_ref, fs_ref):
        e = e_ref[...]
        s3 = jnp.sum(e, axis=0)
        sq3 = jnp.sum(e * e, axis=0)
        d16 = d16_ref[...]
        fm2 = 0.5 * (jnp.dot(s3 * s3, d16, preferred_element_type=jnp.float32)
                     - jnp.dot(sq3, d16, preferred_element_type=jnp.float32))
        h1 = jnp.dot(e[0], w1_ref[0], preferred_element_type=jnp.float32)
        for f in range(1, FIELD):
            h1 = h1 + jnp.dot(e[f], w1_ref[f],
                              preferred_element_type=jnp.float32)
        h1 = jnp.maximum(h1 + b1_ref[...], 0.0)
        h2 = jnp.dot(h1, w2_ref[...], preferred_element_type=jnp.float32)
        h2 = jnp.maximum(h2 + b2_ref[...], 0.0)
        deep = jnp.dot(h2, d8_ref[...], preferred_element_type=jnp.float32)
        fd_ref[...] = fm2 + deep
        fs_ref[...] = jnp.sum(f_ref[...], axis=0)

    return pl.pallas_call(
        body,
        grid=(grid,),
        in_specs=[
            pl.BlockSpec((FIELD, rows, 128), lambda i: (0, i, 0)),
            pl.BlockSpec((FIELD, blk // 128, 128), lambda i: (0, i, 0)),
            pl.BlockSpec((FIELD, 128, 96), lambda i: (0, 0, 0)),
            pl.BlockSpec((96,), lambda i: (0,)),
            pl.BlockSpec((96, 64), lambda i: (0, 0)),
            pl.BlockSpec((64,), lambda i: (0,)),
            pl.BlockSpec((128, 8), lambda i: (0, 0)),
            pl.BlockSpec((64, 8), lambda i: (0, 0)),
        ],
        out_specs=[
            pl.BlockSpec((rows, 8), lambda i: (i, 0)),
            pl.BlockSpec((blk // 128, 128), lambda i: (i, 0)),
        ],
        out_shape=[
            jax.ShapeDtypeStruct((BATCH // 8, 8), jnp.float32),
            jax.ShapeDtypeStruct((BATCH // 128, 128), jnp.float32),
        ],
    )(emb3d, first3d, w1k, b1d, w2d, b2d, d16, d8)


def kernel(x, fm_first_w, fm_second_w, lin1_w, lin1_b, bn1_g, bn1_b,
           lin2_w, lin2_b, bn2_g, bn2_b, bias):
    idx2d = _tc_idx(x.T)
    out1 = _tc_format1(fm_first_w.T)
    pidx2d, first2 = _sc_first(idx2d, out1.reshape(VP))
    out2 = _tc_format2(fm_second_w.T)
    emb3 = _sc_emb(pidx2d, out2.reshape(VP, EMB))
    emb3d = emb3.reshape(FIELD, BATCH * EMB // 128, 128)
    first3d = first2.reshape(FIELD, BATCH // 128, 128)

    a1 = bn1_g / jnp.sqrt(1.0 + EPS)
    w1f = (lin1_w * a1[None, :]).reshape(FIELD, EMB, 12)
    b1f = lin1_b * a1 + bn1_b
    a2 = bn2_g / jnp.sqrt(1.0 + EPS)
    w2f = lin2_w * a2[None, :]
    b2f = lin2_b * a2 + bn2_b
    eye8 = jnp.eye(8, dtype=jnp.float32)
    w1k = jnp.einsum('ab,fej->faebj', eye8, w1f).reshape(FIELD, 128, 96)
    b1d = jnp.tile(b1f, 8)
    w2d = jnp.einsum('ab,ej->aebj', eye8, w2f).reshape(96, 64)
    b2d = jnp.tile(b2f, 8)
    d16 = jnp.einsum('ab,e->aeb', eye8, jnp.ones(EMB)).reshape(128, 8)
    d8 = jnp.einsum('ab,e->aeb', eye8, jnp.ones(8)).reshape(64, 8)

    fd, fs = _tc_dense(emb3d, first3d, w1k, b1d, w2d, b2d, d16, d8)
    tot = fd.reshape(BATCH) + fs.reshape(BATCH) + bias[0]
    return jax.nn.sigmoid(tot)


# speedup vs baseline: 7.6148x; 1.1558x over previous
import functools

import jax
import jax.numpy as jnp
from jax import lax
from jax.experimental import pallas as pl
from jax.experimental.pallas import tpu as pltpu
from jax.experimental.pallas import tpu_sc as plsc

FIELD = 26
EMB = 16
BATCH = 16384
VOCAB = 26 * 38462
BF = BATCH * FIELD
NW = 32
ROWS_PER_W = BF // (NW * 128)
SUB = 13
NGRP = ROWS_PER_W // SUB
EPS = 1e-5
FBLK = 16384
NSUP = (VOCAB + FBLK - 1) // FBLK
VP = NSUP * FBLK


def _tc_idx(xT):
    def body(x_ref, out_ref):
        for f in range(FIELD):
            out_ref[pl.ds(f * (BATCH // 128), BATCH // 128), :] = (
                x_ref[f:f + 1, :].reshape(BATCH // 128, 128))

    return pl.pallas_call(
        body,
        grid=(1,),
        in_specs=[pl.BlockSpec((FIELD, BATCH), lambda f: (0, 0))],
        out_specs=pl.BlockSpec((BF // 128, 128), lambda f: (0, 0)),
        out_shape=jax.ShapeDtypeStruct((BF // 128, 128), jnp.int32),
    )(xT)


def _tc_format1(firstT):
    fblk1 = VP // 8

    def body(f_ref, out1_ref):
        out1_ref[...] = f_ref[...].reshape(fblk1 // 128, 128)

    return pl.pallas_call(
        body,
        grid=(8,),
        in_specs=[pl.BlockSpec((1, fblk1), lambda j: (0, j))],
        out_specs=pl.BlockSpec((fblk1 // 128, 128), lambda j: (j, 0)),
        out_shape=jax.ShapeDtypeStruct((VP // 128, 128), jnp.float32),
    )(firstT)


def _tc_format2(tabT):
    def body(t_ref, out2_ref):
        eye = jax.lax.broadcasted_iota(jnp.int32, (128, 128), 0)
        eye = jnp.where(eye == jax.lax.broadcasted_iota(
            jnp.int32, (128, 128), 1), 1.0, 0.0).astype(jnp.float32)
        for w in range(FBLK // 1024):
            s = jnp.concatenate(
                [t_ref[:, w * 1024 + m * 128: w * 1024 + (m + 1) * 128]
                 for m in range(8)], axis=0)
            r = jax.lax.dot_general(
                s, eye, (((0,), (0,)), ((), ())),
                preferred_element_type=jnp.float32)
            out2_ref[pl.ds(w * 128, 128), :] = r

    return pl.pallas_call(
        body,
        grid=(NSUP,),
        in_specs=[pl.BlockSpec((EMB, FBLK), lambda j: (0, j))],
        out_specs=pl.BlockSpec((FBLK * EMB // 128, 128), lambda j: (j, 0)),
        out_shape=jax.ShapeDtypeStruct((VP * EMB // 128, 128), jnp.float32),
    )(tabT)


def _sc_first(idx2d, tab1):
    mesh = plsc.VectorSubcoreMesh(core_axis_name="c", subcore_axis_name="s")

    @functools.partial(
        pl.kernel,
        out_type=[
            jax.ShapeDtypeStruct((BF // 128, 128), jnp.int32),
            jax.ShapeDtypeStruct((BF // 128, 128), jnp.float32),
        ],
        mesh=mesh,
        compiler_params=pltpu.CompilerParams(use_tc_tiling_on_sc=False),
        scratch_types=[
            pltpu.VMEM((ROWS_PER_W, 128), jnp.int32),
            pltpu.VMEM((ROWS_PER_W, 128), jnp.int32),
            pltpu.VMEM((ROWS_PER_W, 128), jnp.float32),
            pltpu.SemaphoreType.DMA,
        ],
    )
    def k(idx_hbm, tab1_hbm, pidx_out, first_out, idx_v, pidx_v, fv, sem):
        wid = lax.axis_index("s") * 2 + lax.axis_index("c")
        row0 = wid * ROWS_PER_W
        pltpu.sync_copy(idx_hbm.at[pl.ds(row0, ROWS_PER_W)], idx_v)

        def xform(i, carry):
            for s in range(8):
                v = idx_v[i, pl.ds(s * 16, 16)]
                p = (v & -1024) + ((v & 127) << 3) + ((v >> 7) & 7)
                pidx_v[i, pl.ds(s * 16, 16)] = p
            return carry

        lax.fori_loop(0, ROWS_PER_W, xform, 0)
        pltpu.sync_copy(pidx_v, pidx_out.at[pl.ds(row0, ROWS_PER_W)])

        def first_grp(g, carry):
            handles = []
            for b in range(SUB):
                handles.append(pltpu.async_copy(
                    tab1_hbm.at[idx_v.at[g * SUB + b]],
                    fv.at[g * SUB + b], sem))
            for h in handles:
                h.wait()
            return carry

        lax.fori_loop(0, NGRP, first_grp, 0)
        pltpu.sync_copy(fv, first_out.at[pl.ds(row0, ROWS_PER_W)])

    return k(idx2d, tab1)


def _sc_emb(pidx2d, tab2):
    mesh = plsc.VectorSubcoreMesh(core_axis_name="c", subcore_axis_name="s")

    @functools.partial(
        pl.kernel,
        out_type=jax.ShapeDtypeStruct((BF // 128, 128, EMB), jnp.float32),
        mesh=mesh,
        compiler_params=pltpu.CompilerParams(use_tc_tiling_on_sc=False),
        scratch_types=[
            pltpu.VMEM((ROWS_PER_W, 128), jnp.int32),
            pltpu.VMEM((SUB, 128, EMB), jnp.float32),
            pltpu.SemaphoreType.DMA,
        ],
    )
    def k(pidx_hbm, tab2_hbm, emb_out, pidx_v, rows_v, sem):
        wid = lax.axis_index("s") * 2 + lax.axis_index("c")
        row0 = wid * ROWS_PER_W
        pltpu.sync_copy(pidx_hbm.at[pl.ds(row0, ROWS_PER_W)], pidx_v)

        def emb_grp(g, carry):
            handles = []
            for b in range(SUB):
                handles.append(pltpu.async_copy(
                    tab2_hbm.at[pidx_v.at[g * SUB + b]], rows_v.at[b], sem))
            for h in handles:
                h.wait()
            pltpu.sync_copy(rows_v, emb_out.at[pl.ds(row0 + g * SUB, SUB)])
            return carry

        lax.fori_loop(0, NGRP, emb_grp, 0)

    return k(pidx2d, tab2)


def _tc_dense(emb3d, first3d, w1k, b1d, w2d, b2d, d16, d8):
    blk = 1024
    grid = BATCH // blk
    rows = blk * EMB // 128

    def body(e_ref, f_ref, w1_ref, b1_ref, w2_ref, b2_ref, d16_ref, d8_ref,
             fd_ref, fs_ref):
        e = e_ref[...]
        s3 = jnp.sum(e, axis=0)
        sq3 = jnp.sum(e * e, axis=0)
        d16 = d16_ref[...]
        fm2 = 0.5 * (jnp.dot(s3 * s3, d16, preferred_element_type=jnp.float32)
                     - jnp.dot(sq3, d16, preferred_element_type=jnp.float32))
        h1 = jnp.dot(e[0], w1_ref[0], preferred_element_type=jnp.float32)
        for f in range(1, FIELD):
            h1 = h1 + jnp.dot(e[f], w1_ref[f],
                              preferred_element_type=jnp.float32)
        h1 = jnp.maximum(h1 + b1_ref[...], 0.0)
        h2 = jnp.dot(h1, w2_ref[...], preferred_element_type=jnp.float32)
        h2 = jnp.maximum(h2 + b2_ref[...], 0.0)
        deep = jnp.dot(h2, d8_ref[...], preferred_element_type=jnp.float32)
        fd_ref[...] = fm2 + deep
        fs_ref[...] = jnp.sum(f_ref[...], axis=0)

    return pl.pallas_call(
        body,
        grid=(grid,),
        in_specs=[
            pl.BlockSpec((FIELD, rows, 128), lambda i: (0, i, 0)),
            pl.BlockSpec((FIELD, blk // 128, 128), lambda i: (0, i, 0)),
            pl.BlockSpec((FIELD, 128, 96), lambda i: (0, 0, 0)),
            pl.BlockSpec((96,), lambda i: (0,)),
            pl.BlockSpec((96, 64), lambda i: (0, 0)),
            pl.BlockSpec((64,), lambda i: (0,)),
            pl.BlockSpec((128, 8), lambda i: (0, 0)),
            pl.BlockSpec((64, 8), lambda i: (0, 0)),
        ],
        out_specs=[
            pl.BlockSpec((rows, 8), lambda i: (i, 0)),
            pl.BlockSpec((blk // 128, 128), lambda i: (i, 0)),
        ],
        out_shape=[
            jax.ShapeDtypeStruct((BATCH // 8, 8), jnp.float32),
            jax.ShapeDtypeStruct((BATCH // 128, 128), jnp.float32),
        ],
    )(emb3d, first3d, w1k, b1d, w2d, b2d, d16, d8)


def kernel(x, fm_first_w, fm_second_w, lin1_w, lin1_b, bn1_g, bn1_b,
           lin2_w, lin2_b, bn2_g, bn2_b, bias):
    idx2d = _tc_idx(x.T)
    out1 = _tc_format1(fm_first_w.T)
    pidx2d, first2 = _sc_first(idx2d, out1.reshape(VP))
    out2 = _tc_format2(fm_second_w.T)
    emb3 = _sc_emb(pidx2d, out2.reshape(VP, EMB))
    emb3d = emb3.reshape(FIELD, BATCH * EMB // 128, 128)
    first3d = first2.reshape(FIELD, BATCH // 128, 128)

    a1 = bn1_g / jnp.sqrt(1.0 + EPS)
    w1f = (lin1_w * a1[None, :]).reshape(FIELD, EMB, 12)
    b1f = lin1_b * a1 + bn1_b
    a2 = bn2_g / jnp.sqrt(1.0 + EPS)
    w2f = lin2_w * a2[None, :]
    b2f = lin2_b * a2 + bn2_b
    eye8 = jnp.eye(8, dtype=jnp.float32)
    w1k = jnp.einsum('ab,fej->faebj', eye8, w1f).reshape(FIELD, 128, 96)
    b1d = jnp.tile(b1f, 8)
    w2d = jnp.einsum('ab,ej->aebj', eye8, w2f).reshape(96, 64)
    b2d = jnp.tile(b2f, 8)
    d16 = jnp.einsum('ab,e->aeb', eye8, jnp.ones(EMB)).reshape(128, 8)
    d8 = jnp.einsum('ab,e->aeb', eye8, jnp.ones(8)).reshape(64, 8)

    fd, fs = _tc_dense(emb3d, first3d, w1k, b1d, w2d, b2d, d16, d8)
    tot = fd.reshape(BATCH) + fs.reshape(BATCH) + bias[0]
    return jax.nn.sigmoid(tot)


# speedup vs baseline: 8.4647x; 1.1116x over previous
import functools

import jax
import jax.numpy as jnp
from jax import lax
from jax.experimental import pallas as pl
from jax.experimental.pallas import tpu as pltpu
from jax.experimental.pallas import tpu_sc as plsc

FIELD = 26
EMB = 16
BATCH = 16384
VOCAB = 26 * 38462
BF = BATCH * FIELD
NW = 32
ROWS_PER_W = BF // (NW * 128)
SUB = 13
NGRP = ROWS_PER_W // SUB
EPS = 1e-5
FBLK = 32768
NSUP = (VOCAB + FBLK - 1) // FBLK
VP = NSUP * FBLK


def _tc_idx(xT):
    def body(x_ref, out_ref):
        for f in range(FIELD):
            out_ref[pl.ds(f * (BATCH // 128), BATCH // 128), :] = (
                x_ref[f:f + 1, :].reshape(BATCH // 128, 128))

    return pl.pallas_call(
        body,
        grid=(1,),
        in_specs=[pl.BlockSpec((FIELD, BATCH), lambda f: (0, 0))],
        out_specs=pl.BlockSpec((BF // 128, 128), lambda f: (0, 0)),
        out_shape=jax.ShapeDtypeStruct((BF // 128, 128), jnp.int32),
    )(xT)


def _tc_format1(firstT):
    fblk1 = VP // 8

    def body(f_ref, out1_ref):
        out1_ref[...] = f_ref[...].reshape(fblk1 // 128, 128)

    return pl.pallas_call(
        body,
        grid=(8,),
        in_specs=[pl.BlockSpec((1, fblk1), lambda j: (0, j))],
        out_specs=pl.BlockSpec((fblk1 // 128, 128), lambda j: (j, 0)),
        out_shape=jax.ShapeDtypeStruct((VP // 128, 128), jnp.float32),
    )(firstT)


def _tc_format2(tabT):
    def body(t_ref, out2_ref):
        eye = jax.lax.broadcasted_iota(jnp.int32, (128, 128), 0)
        eye = jnp.where(eye == jax.lax.broadcasted_iota(
            jnp.int32, (128, 128), 1), 1.0, 0.0).astype(jnp.float32)
        for w in range(FBLK // 1024):
            s = jnp.concatenate(
                [t_ref[:, w * 1024 + m * 128: w * 1024 + (m + 1) * 128]
                 for m in range(8)], axis=0)
            r = jax.lax.dot_general(
                s, eye, (((0,), (0,)), ((), ())),
                preferred_element_type=jnp.float32)
            out2_ref[pl.ds(w * 128, 128), :] = r

    return pl.pallas_call(
        body,
        grid=(NSUP,),
        in_specs=[pl.BlockSpec((EMB, FBLK), lambda j: (0, j))],
        out_specs=pl.BlockSpec((FBLK * EMB // 128, 128), lambda j: (j, 0)),
        out_shape=jax.ShapeDtypeStruct((VP * EMB // 128, 128), jnp.float32),
    )(tabT)


def _sc_first(idx2d, tab1):
    mesh = plsc.VectorSubcoreMesh(core_axis_name="c", subcore_axis_name="s")

    @functools.partial(
        pl.kernel,
        out_type=[
            jax.ShapeDtypeStruct((BF // 128, 128), jnp.int32),
            jax.ShapeDtypeStruct((BF // 128, 128), jnp.float32),
        ],
        mesh=mesh,
        compiler_params=pltpu.CompilerParams(use_tc_tiling_on_sc=False),
        scratch_types=[
            pltpu.VMEM((ROWS_PER_W, 128), jnp.int32),
            pltpu.VMEM((ROWS_PER_W, 128), jnp.int32),
            pltpu.VMEM((ROWS_PER_W, 128), jnp.float32),
            pltpu.SemaphoreType.DMA,
        ],
    )
    def k(idx_hbm, tab1_hbm, pidx_out, first_out, idx_v, pidx_v, fv, sem):
        wid = lax.axis_index("s") * 2 + lax.axis_index("c")
        row0 = wid * ROWS_PER_W
        pltpu.sync_copy(idx_hbm.at[pl.ds(row0, ROWS_PER_W)], idx_v)

        def xform(i, carry):
            for s in range(8):
                v = idx_v[i, pl.ds(s * 16, 16)]
                p = (v & -1024) + ((v & 127) << 3) + ((v >> 7) & 7)
                pidx_v[i, pl.ds(s * 16, 16)] = p
            return carry

        lax.fori_loop(0, ROWS_PER_W, xform, 0)
        pltpu.sync_copy(pidx_v, pidx_out.at[pl.ds(row0, ROWS_PER_W)])

        def first_grp(g, carry):
            handles = []
            for b in range(SUB):
                handles.append(pltpu.async_copy(
                    tab1_hbm.at[idx_v.at[g * SUB + b]],
                    fv.at[g * SUB + b], sem))
            for h in handles:
                h.wait()
            return carry

        lax.fori_loop(0, NGRP, first_grp, 0)
        pltpu.sync_copy(fv, first_out.at[pl.ds(row0, ROWS_PER_W)])

    return k(idx2d, tab1)


def _sc_emb(pidx2d, tab2):
    mesh = plsc.VectorSubcoreMesh(core_axis_name="c", subcore_axis_name="s")

    @functools.partial(
        pl.kernel,
        out_type=jax.ShapeDtypeStruct((BF // 128, 128, EMB), jnp.float32),
        mesh=mesh,
        compiler_params=pltpu.CompilerParams(use_tc_tiling_on_sc=False),
        scratch_types=[
            pltpu.VMEM((ROWS_PER_W, 128), jnp.int32),
            pltpu.VMEM((SUB, 128, EMB), jnp.float32),
            pltpu.SemaphoreType.DMA,
        ],
    )
    def k(pidx_hbm, tab2_hbm, emb_out, pidx_v, rows_v, sem):
        wid = lax.axis_index("s") * 2 + lax.axis_index("c")
        row0 = wid * ROWS_PER_W
        pltpu.sync_copy(pidx_hbm.at[pl.ds(row0, ROWS_PER_W)], pidx_v)

        def emb_grp(g, carry):
            handles = []
            for b in range(SUB):
                handles.append(pltpu.async_copy(
                    tab2_hbm.at[pidx_v.at[g * SUB + b]], rows_v.at[b], sem))
            for h in handles:
                h.wait()
            pltpu.sync_copy(rows_v, emb_out.at[pl.ds(row0 + g * SUB, SUB)])
            return carry

        lax.fori_loop(0, NGRP, emb_grp, 0)

    return k(pidx2d, tab2)


def _tc_dense(emb3d, first3d, w1k, b1d, w2d, b2d, d16, d8):
    blk = 1024
    grid = BATCH // blk
    rows = blk * EMB // 128

    def body(e_ref, f_ref, w1_ref, b1_ref, w2_ref, b2_ref, d16_ref, d8_ref,
             fd_ref, fs_ref):
        e = e_ref[...]
        s3 = jnp.sum(e, axis=0)
        sq3 = jnp.sum(e * e, axis=0)
        d16 = d16_ref[...]
        fm2 = 0.5 * (jnp.dot(s3 * s3, d16, preferred_element_type=jnp.float32)
                     - jnp.dot(sq3, d16, preferred_element_type=jnp.float32))
        h1 = jnp.dot(e[0], w1_ref[0], preferred_element_type=jnp.float32)
        for f in range(1, FIELD):
            h1 = h1 + jnp.dot(e[f], w1_ref[f],
                              preferred_element_type=jnp.float32)
        h1 = jnp.maximum(h1 + b1_ref[...], 0.0)
        h2 = jnp.dot(h1, w2_ref[...], preferred_element_type=jnp.float32)
        h2 = jnp.maximum(h2 + b2_ref[...], 0.0)
        deep = jnp.dot(h2, d8_ref[...], preferred_element_type=jnp.float32)
        fd_ref[...] = fm2 + deep
        fs_ref[...] = jnp.sum(f_ref[...], axis=0)

    return pl.pallas_call(
        body,
        grid=(grid,),
        in_specs=[
            pl.BlockSpec((FIELD, rows, 128), lambda i: (0, i, 0)),
            pl.BlockSpec((FIELD, blk // 128, 128), lambda i: (0, i, 0)),
            pl.BlockSpec((FIELD, 128, 96), lambda i: (0, 0, 0)),
            pl.BlockSpec((96,), lambda i: (0,)),
            pl.BlockSpec((96, 64), lambda i: (0, 0)),
            pl.BlockSpec((64,), lambda i: (0,)),
            pl.BlockSpec((128, 8), lambda i: (0, 0)),
            pl.BlockSpec((64, 8), lambda i: (0, 0)),
        ],
        out_specs=[
            pl.BlockSpec((rows, 8), lambda i: (i, 0)),
            pl.BlockSpec((blk // 128, 128), lambda i: (i, 0)),
        ],
        out_shape=[
            jax.ShapeDtypeStruct((BATCH // 8, 8), jnp.float32),
            jax.ShapeDtypeStruct((BATCH // 128, 128), jnp.float32),
        ],
    )(emb3d, first3d, w1k, b1d, w2d, b2d, d16, d8)


def kernel(x, fm_first_w, fm_second_w, lin1_w, lin1_b, bn1_g, bn1_b,
           lin2_w, lin2_b, bn2_g, bn2_b, bias):
    idx2d = _tc_idx(x.T)
    out1 = _tc_format1(fm_first_w.T)
    pidx2d, first2 = _sc_first(idx2d, out1.reshape(VP))
    out2 = _tc_format2(fm_second_w.T)
    emb3 = _sc_emb(pidx2d, out2.reshape(VP, EMB))
    emb3d = emb3.reshape(FIELD, BATCH * EMB // 128, 128)
    first3d = first2.reshape(FIELD, BATCH // 128, 128)

    a1 = bn1_g / jnp.sqrt(1.0 + EPS)
    w1f = (lin1_w * a1[None, :]).reshape(FIELD, EMB, 12)
    b1f = lin1_b * a1 + bn1_b
    a2 = bn2_g / jnp.sqrt(1.0 + EPS)
    w2f = lin2_w * a2[None, :]
    b2f = lin2_b * a2 + bn2_b
    eye8 = jnp.eye(8, dtype=jnp.float32)
    w1k = jnp.einsum('ab,fej->faebj', eye8, w1f).reshape(FIELD, 128, 96)
    b1d = jnp.tile(b1f, 8)
    w2d = jnp.einsum('ab,ej->aebj', eye8, w2f).reshape(96, 64)
    b2d = jnp.tile(b2f, 8)
    d16 = jnp.einsum('ab,e->aeb', eye8, jnp.ones(EMB)).reshape(128, 8)
    d8 = jnp.einsum('ab,e->aeb', eye8, jnp.ones(8)).reshape(64, 8)

    fd, fs = _tc_dense(emb3d, first3d, w1k, b1d, w2d, b2d, d16, d8)
    tot = fd.reshape(BATCH) + fs.reshape(BATCH) + bias[0]
    return jax.nn.sigmoid(tot)


# speedup vs baseline: 8.6619x; 1.0233x over previous
import functools

import jax
import jax.numpy as jnp
from jax import lax
from jax.experimental import pallas as pl
from jax.experimental.pallas import tpu as pltpu
from jax.experimental.pallas import tpu_sc as plsc

FIELD = 26
EMB = 16
BATCH = 16384
VOCAB = 26 * 38462
BF = BATCH * FIELD
NW = 32
ROWS_PER_W = BF // (NW * 128)
SUB = 13
NGRP = ROWS_PER_W // SUB
EPS = 1e-5
FBLK = 65536
NSUP = (VOCAB + FBLK - 1) // FBLK
VP = NSUP * FBLK


def _tc_idx(xT):
    def body(x_ref, out_ref):
        for f in range(FIELD):
            out_ref[pl.ds(f * (BATCH // 128), BATCH // 128), :] = (
                x_ref[f:f + 1, :].reshape(BATCH // 128, 128))

    return pl.pallas_call(
        body,
        grid=(1,),
        in_specs=[pl.BlockSpec((FIELD, BATCH), lambda f: (0, 0))],
        out_specs=pl.BlockSpec((BF // 128, 128), lambda f: (0, 0)),
        out_shape=jax.ShapeDtypeStruct((BF // 128, 128), jnp.int32),
    )(xT)


def _tc_format1(firstT):
    fblk1 = VP // 8

    def body(f_ref, out1_ref):
        out1_ref[...] = f_ref[...].reshape(fblk1 // 128, 128)

    return pl.pallas_call(
        body,
        grid=(8,),
        in_specs=[pl.BlockSpec((1, fblk1), lambda j: (0, j))],
        out_specs=pl.BlockSpec((fblk1 // 128, 128), lambda j: (j, 0)),
        out_shape=jax.ShapeDtypeStruct((VP // 128, 128), jnp.float32),
    )(firstT)


def _tc_format2(tabT):
    def body(t_ref, out2_ref):
        eye = jax.lax.broadcasted_iota(jnp.int32, (128, 128), 0)
        eye = jnp.where(eye == jax.lax.broadcasted_iota(
            jnp.int32, (128, 128), 1), 1.0, 0.0).astype(jnp.float32)
        for w in range(FBLK // 1024):
            s = jnp.concatenate(
                [t_ref[:, w * 1024 + m * 128: w * 1024 + (m + 1) * 128]
                 for m in range(8)], axis=0)
            r = jax.lax.dot_general(
                s, eye, (((0,), (0,)), ((), ())),
                preferred_element_type=jnp.float32)
            out2_ref[pl.ds(w * 128, 128), :] = r

    return pl.pallas_call(
        body,
        grid=(NSUP,),
        in_specs=[pl.BlockSpec((EMB, FBLK), lambda j: (0, j))],
        out_specs=pl.BlockSpec((FBLK * EMB // 128, 128), lambda j: (j, 0)),
        out_shape=jax.ShapeDtypeStruct((VP * EMB // 128, 128), jnp.float32),
    )(tabT)


def _sc_first(idx2d, tab1):
    mesh = plsc.VectorSubcoreMesh(core_axis_name="c", subcore_axis_name="s")

    @functools.partial(
        pl.kernel,
        out_type=[
            jax.ShapeDtypeStruct((BF // 128, 128), jnp.int32),
            jax.ShapeDtypeStruct((BF // 128, 128), jnp.float32),
        ],
        mesh=mesh,
        compiler_params=pltpu.CompilerParams(use_tc_tiling_on_sc=False),
        scratch_types=[
            pltpu.VMEM((ROWS_PER_W, 128), jnp.int32),
            pltpu.VMEM((ROWS_PER_W, 128), jnp.int32),
            pltpu.VMEM((ROWS_PER_W, 128), jnp.float32),
            pltpu.SemaphoreType.DMA,
        ],
    )
    def k(idx_hbm, tab1_hbm, pidx_out, first_out, idx_v, pidx_v, fv, sem):
        wid = lax.axis_index("s") * 2 + lax.axis_index("c")
        row0 = wid * ROWS_PER_W
        pltpu.sync_copy(idx_hbm.at[pl.ds(row0, ROWS_PER_W)], idx_v)

        def xform(i, carry):
            for s in range(8):
                v = idx_v[i, pl.ds(s * 16, 16)]
                p = (v & -1024) + ((v & 127) << 3) + ((v >> 7) & 7)
                pidx_v[i, pl.ds(s * 16, 16)] = p
            return carry

        lax.fori_loop(0, ROWS_PER_W, xform, 0)
        pltpu.sync_copy(pidx_v, pidx_out.at[pl.ds(row0, ROWS_PER_W)])

        def first_grp(g, carry):
            handles = []
            for b in range(SUB):
                handles.append(pltpu.async_copy(
                    tab1_hbm.at[idx_v.at[g * SUB + b]],
                    fv.at[g * SUB + b], sem))
            for h in handles:
                h.wait()
            return carry

        lax.fori_loop(0, NGRP, first_grp, 0)
        pltpu.sync_copy(fv, first_out.at[pl.ds(row0, ROWS_PER_W)])

    return k(idx2d, tab1)


def _sc_emb(pidx2d, tab2):
    mesh = plsc.VectorSubcoreMesh(core_axis_name="c", subcore_axis_name="s")

    @functools.partial(
        pl.kernel,
        out_type=jax.ShapeDtypeStruct((BF // 128, 128, EMB), jnp.float32),
        mesh=mesh,
        compiler_params=pltpu.CompilerParams(use_tc_tiling_on_sc=False),
        scratch_types=[
            pltpu.VMEM((ROWS_PER_W, 128), jnp.int32),
            pltpu.VMEM((SUB, 128, EMB), jnp.float32),
            pltpu.SemaphoreType.DMA,
        ],
    )
    def k(pidx_hbm, tab2_hbm, emb_out, pidx_v, rows_v, sem):
        wid = lax.axis_index("s") * 2 + lax.axis_index("c")
        row0 = wid * ROWS_PER_W
        pltpu.sync_copy(pidx_hbm.at[pl.ds(row0, ROWS_PER_W)], pidx_v)

        def emb_grp(g, carry):
            handles = []
            for b in range(SUB):
                handles.append(pltpu.async_copy(
                    tab2_hbm.at[pidx_v.at[g * SUB + b]], rows_v.at[b], sem))
            for h in handles:
                h.wait()
            pltpu.sync_copy(rows_v, emb_out.at[pl.ds(row0 + g * SUB, SUB)])
            return carry

        lax.fori_loop(0, NGRP, emb_grp, 0)

    return k(pidx2d, tab2)


def _tc_dense(emb3d, first3d, w1k, b1d, w2d, b2d, d16, d8):
    blk = 1024
    grid = BATCH // blk
    rows = blk * EMB // 128

    def body(e_ref, f_ref, w1_ref, b1_ref, w2_ref, b2_ref, d16_ref, d8_ref,
             fd_ref, fs_ref):
        e = e_ref[...]
        s3 = jnp.sum(e, axis=0)
        sq3 = jnp.sum(e * e, axis=0)
        d16 = d16_ref[...]
        fm2 = 0.5 * (jnp.dot(s3 * s3, d16, preferred_element_type=jnp.float32)
                     - jnp.dot(sq3, d16, preferred_element_type=jnp.float32))
        h1 = jnp.dot(e[0], w1_ref[0], preferred_element_type=jnp.float32)
        for f in range(1, FIELD):
            h1 = h1 + jnp.dot(e[f], w1_ref[f],
                              preferred_element_type=jnp.float32)
        h1 = jnp.maximum(h1 + b1_ref[...], 0.0)
        h2 = jnp.dot(h1, w2_ref[...], preferred_element_type=jnp.float32)
        h2 = jnp.maximum(h2 + b2_ref[...], 0.0)
        deep = jnp.dot(h2, d8_ref[...], preferred_element_type=jnp.float32)
        fd_ref[...] = fm2 + deep
        fs_ref[...] = jnp.sum(f_ref[...], axis=0)

    return pl.pallas_call(
        body,
        grid=(grid,),
        in_specs=[
            pl.BlockSpec((FIELD, rows, 128), lambda i: (0, i, 0)),
            pl.BlockSpec((FIELD, blk // 128, 128), lambda i: (0, i, 0)),
            pl.BlockSpec((FIELD, 128, 96), lambda i: (0, 0, 0)),
            pl.BlockSpec((96,), lambda i: (0,)),
            pl.BlockSpec((96, 64), lambda i: (0, 0)),
            pl.BlockSpec((64,), lambda i: (0,)),
            pl.BlockSpec((128, 8), lambda i: (0, 0)),
            pl.BlockSpec((64, 8), lambda i: (0, 0)),
        ],
        out_specs=[
            pl.BlockSpec((rows, 8), lambda i: (i, 0)),
            pl.BlockSpec((blk // 128, 128), lambda i: (i, 0)),
        ],
        out_shape=[
            jax.ShapeDtypeStruct((BATCH // 8, 8), jnp.float32),
            jax.ShapeDtypeStruct((BATCH // 128, 128), jnp.float32),
        ],
    )(emb3d, first3d, w1k, b1d, w2d, b2d, d16, d8)


def kernel(x, fm_first_w, fm_second_w, lin1_w, lin1_b, bn1_g, bn1_b,
           lin2_w, lin2_b, bn2_g, bn2_b, bias):
    idx2d = _tc_idx(x.T)
    out1 = _tc_format1(fm_first_w.T)
    pidx2d, first2 = _sc_first(idx2d, out1.reshape(VP))
    out2 = _tc_format2(fm_second_w.T)
    emb3 = _sc_emb(pidx2d, out2.reshape(VP, EMB))
    emb3d = emb3.reshape(FIELD, BATCH * EMB // 128, 128)
    first3d = first2.reshape(FIELD, BATCH // 128, 128)

    a1 = bn1_g / jnp.sqrt(1.0 + EPS)
    w1f = (lin1_w * a1[None, :]).reshape(FIELD, EMB, 12)
    b1f = lin1_b * a1 + bn1_b
    a2 = bn2_g / jnp.sqrt(1.0 + EPS)
    w2f = lin2_w * a2[None, :]
    b2f = lin2_b * a2 + bn2_b
    eye8 = jnp.eye(8, dtype=jnp.float32)
    w1k = jnp.einsum('ab,fej->faebj', eye8, w1f).reshape(FIELD, 128, 96)
    b1d = jnp.tile(b1f, 8)
    w2d = jnp.einsum('ab,ej->aebj', eye8, w2f).reshape(96, 64)
    b2d = jnp.tile(b2f, 8)
    d16 = jnp.einsum('ab,e->aeb', eye8, jnp.ones(EMB)).reshape(128, 8)
    d8 = jnp.einsum('ab,e->aeb', eye8, jnp.ones(8)).reshape(64, 8)

    fd, fs = _tc_dense(emb3d, first3d, w1k, b1d, w2d, b2d, d16, d8)
    tot = fd.reshape(BATCH) + fs.reshape(BATCH) + bias[0]
    return jax.nn.sigmoid(tot)


# speedup vs baseline: 8.9174x; 1.0295x over previous
import functools

import jax
import jax.numpy as jnp
from jax import lax
from jax.experimental import pallas as pl
from jax.experimental.pallas import tpu as pltpu
from jax.experimental.pallas import tpu_sc as plsc

FIELD = 26
EMB = 16
BATCH = 16384
VOCAB = 26 * 38462
BF = BATCH * FIELD
NW = 32
ROWS_PER_W = BF // (NW * 128)
SUB = 13
NGRP = ROWS_PER_W // SUB
EPS = 1e-5
FBLK = 65536
NSUP = (VOCAB + FBLK - 1) // FBLK
VP = NSUP * FBLK


def _tc_idx(xT):
    def body(x_ref, out_ref):
        for f in range(FIELD):
            out_ref[pl.ds(f * (BATCH // 128), BATCH // 128), :] = (
                x_ref[f:f + 1, :].reshape(BATCH // 128, 128))

    return pl.pallas_call(
        body,
        grid=(1,),
        in_specs=[pl.BlockSpec((FIELD, BATCH), lambda f: (0, 0))],
        out_specs=pl.BlockSpec((BF // 128, 128), lambda f: (0, 0)),
        out_shape=jax.ShapeDtypeStruct((BF // 128, 128), jnp.int32),
    )(xT)


def _tc_format1(firstT):
    fblk1 = VP // 8

    def body(f_ref, out1_ref):
        out1_ref[...] = f_ref[...].reshape(fblk1 // 128, 128)

    return pl.pallas_call(
        body,
        grid=(8,),
        in_specs=[pl.BlockSpec((1, fblk1), lambda j: (0, j))],
        out_specs=pl.BlockSpec((fblk1 // 128, 128), lambda j: (j, 0)),
        out_shape=jax.ShapeDtypeStruct((VP // 128, 128), jnp.float32),
    )(firstT)


def _tc_format2(tabT):
    def body(t_ref, out2_ref):
        eye = jax.lax.broadcasted_iota(jnp.int32, (128, 128), 0)
        eye = jnp.where(eye == jax.lax.broadcasted_iota(
            jnp.int32, (128, 128), 1), 1.0, 0.0).astype(jnp.float32)
        for w in range(FBLK // 1024):
            s = jnp.concatenate(
                [t_ref[:, w * 1024 + m * 128: w * 1024 + (m + 1) * 128]
                 for m in range(8)], axis=0)
            r = jax.lax.dot_general(
                s, eye, (((0,), (0,)), ((), ())),
                preferred_element_type=jnp.float32)
            out2_ref[pl.ds(w * 128, 128), :] = r

    return pl.pallas_call(
        body,
        grid=(NSUP,),
        in_specs=[pl.BlockSpec((EMB, FBLK), lambda j: (0, j))],
        out_specs=pl.BlockSpec((FBLK * EMB // 128, 128), lambda j: (j, 0)),
        out_shape=jax.ShapeDtypeStruct((VP * EMB // 128, 128), jnp.float32),
    )(tabT)


def _sc_first(idx2d, tab1):
    mesh = plsc.VectorSubcoreMesh(core_axis_name="c", subcore_axis_name="s")

    @functools.partial(
        pl.kernel,
        out_type=[
            jax.ShapeDtypeStruct((BF // 128, 128), jnp.int32),
            jax.ShapeDtypeStruct((BF // 128, 128), jnp.float32),
        ],
        mesh=mesh,
        compiler_params=pltpu.CompilerParams(use_tc_tiling_on_sc=False),
        scratch_types=[
            pltpu.VMEM((ROWS_PER_W, 128), jnp.int32),
            pltpu.VMEM((ROWS_PER_W, 128), jnp.int32),
            pltpu.VMEM((ROWS_PER_W, 128), jnp.float32),
            pltpu.SemaphoreType.DMA,
        ],
    )
    def k(idx_hbm, tab1_hbm, pidx_out, first_out, idx_v, pidx_v, fv, sem):
        wid = lax.axis_index("s") * 2 + lax.axis_index("c")
        row0 = wid * ROWS_PER_W
        pltpu.sync_copy(idx_hbm.at[pl.ds(row0, ROWS_PER_W)], idx_v)

        def xform(i, carry):
            for s in range(8):
                v = idx_v[i, pl.ds(s * 16, 16)]
                p = (v & -1024) + ((v & 127) << 3) + ((v >> 7) & 7)
                pidx_v[i, pl.ds(s * 16, 16)] = p
            return carry

        lax.fori_loop(0, ROWS_PER_W, xform, 0)
        pltpu.sync_copy(pidx_v, pidx_out.at[pl.ds(row0, ROWS_PER_W)])

        def first_grp(g, carry):
            handles = []
            for b in range(SUB):
                handles.append(pltpu.async_copy(
                    tab1_hbm.at[idx_v.at[g * SUB + b]],
                    fv.at[g * SUB + b], sem))
            for h in handles:
                h.wait()
            return carry

        lax.fori_loop(0, NGRP, first_grp, 0)
        pltpu.sync_copy(fv, first_out.at[pl.ds(row0, ROWS_PER_W)])

    return k(idx2d, tab1)


def _sc_emb(pidx2d, tab2):
    mesh = plsc.VectorSubcoreMesh(core_axis_name="c", subcore_axis_name="s")

    @functools.partial(
        pl.kernel,
        out_type=jax.ShapeDtypeStruct((BF // 128, 128, EMB), jnp.float32),
        mesh=mesh,
        compiler_params=pltpu.CompilerParams(use_tc_tiling_on_sc=False),
        scratch_types=[
            pltpu.VMEM((ROWS_PER_W, 128), jnp.int32),
            pltpu.VMEM((SUB, 128, EMB), jnp.float32),
            pltpu.VMEM((SUB, 128, EMB), jnp.float32),
            pltpu.SemaphoreType.DMA,
            pltpu.SemaphoreType.DMA,
        ],
    )
    def k(pidx_hbm, tab2_hbm, emb_out, pidx_v, rows_a, rows_b, sem_a,
          sem_b):
        wid = lax.axis_index("s") * 2 + lax.axis_index("c")
        row0 = wid * ROWS_PER_W
        pltpu.sync_copy(pidx_hbm.at[pl.ds(row0, ROWS_PER_W)], pidx_v)

        def issue(g, buf, sem):
            for b in range(SUB):
                pltpu.async_copy(
                    tab2_hbm.at[pidx_v.at[g * SUB + b]], buf.at[b], sem)

        def drain(buf, sem):
            for b in range(SUB):
                pltpu.make_async_copy(
                    tab2_hbm.at[pidx_v.at[b]], buf.at[b], sem).wait()

        def write(g, buf):
            pltpu.sync_copy(buf, emb_out.at[pl.ds(row0 + g * SUB, SUB)])

        issue(0, rows_a, sem_a)

        def pair(h, carry):
            g = 2 * h
            issue(g + 1, rows_b, sem_b)
            drain(rows_a, sem_a)
            write(g, rows_a)
            issue(g + 2, rows_a, sem_a)
            drain(rows_b, sem_b)
            write(g + 1, rows_b)
            return carry

        lax.fori_loop(0, NGRP // 2 - 1, pair, 0)
        g = NGRP - 2
        issue(g + 1, rows_b, sem_b)
        drain(rows_a, sem_a)
        write(g, rows_a)
        drain(rows_b, sem_b)
        write(g + 1, rows_b)

    return k(pidx2d, tab2)


def _tc_dense(emb3d, first3d, w1k, b1d, w2d, b2d, d16, d8):
    blk = 1024
    grid = BATCH // blk
    rows = blk * EMB // 128

    def body(e_ref, f_ref, w1_ref, b1_ref, w2_ref, b2_ref, d16_ref, d8_ref,
             fd_ref, fs_ref):
        e = e_ref[...]
        s3 = jnp.sum(e, axis=0)
        sq3 = jnp.sum(e * e, axis=0)
        d16 = d16_ref[...]
        fm2 = 0.5 * (jnp.dot(s3 * s3, d16, preferred_element_type=jnp.float32)
                     - jnp.dot(sq3, d16, preferred_element_type=jnp.float32))
        h1 = jnp.dot(e[0], w1_ref[0], preferred_element_type=jnp.float32)
        for f in range(1, FIELD):
            h1 = h1 + jnp.dot(e[f], w1_ref[f],
                              preferred_element_type=jnp.float32)
        h1 = jnp.maximum(h1 + b1_ref[...], 0.0)
        h2 = jnp.dot(h1, w2_ref[...], preferred_element_type=jnp.float32)
        h2 = jnp.maximum(h2 + b2_ref[...], 0.0)
        deep = jnp.dot(h2, d8_ref[...], preferred_element_type=jnp.float32)
        fd_ref[...] = fm2 + deep
        fs_ref[...] = jnp.sum(f_ref[...], axis=0)

    return pl.pallas_call(
        body,
        grid=(grid,),
        in_specs=[
            pl.BlockSpec((FIELD, rows, 128), lambda i: (0, i, 0)),
            pl.BlockSpec((FIELD, blk // 128, 128), lambda i: (0, i, 0)),
            pl.BlockSpec((FIELD, 128, 96), lambda i: (0, 0, 0)),
            pl.BlockSpec((96,), lambda i: (0,)),
            pl.BlockSpec((96, 64), lambda i: (0, 0)),
            pl.BlockSpec((64,), lambda i: (0,)),
            pl.BlockSpec((128, 8), lambda i: (0, 0)),
            pl.BlockSpec((64, 8), lambda i: (0, 0)),
        ],
        out_specs=[
            pl.BlockSpec((rows, 8), lambda i: (i, 0)),
            pl.BlockSpec((blk // 128, 128), lambda i: (i, 0)),
        ],
        out_shape=[
            jax.ShapeDtypeStruct((BATCH // 8, 8), jnp.float32),
            jax.ShapeDtypeStruct((BATCH // 128, 128), jnp.float32),
        ],
    )(emb3d, first3d, w1k, b1d, w2d, b2d, d16, d8)


def kernel(x, fm_first_w, fm_second_w, lin1_w, lin1_b, bn1_g, bn1_b,
           lin2_w, lin2_b, bn2_g, bn2_b, bias):
    idx2d = _tc_idx(x.T)
    out1 = _tc_format1(fm_first_w.T)
    pidx2d, first2 = _sc_first(idx2d, out1.reshape(VP))
    out2 = _tc_format2(fm_second_w.T)
    emb3 = _sc_emb(pidx2d, out2.reshape(VP, EMB))
    emb3d = emb3.reshape(FIELD, BATCH * EMB // 128, 128)
    first3d = first2.reshape(FIELD, BATCH // 128, 128)

    a1 = bn1_g / jnp.sqrt(1.0 + EPS)
    w1f = (lin1_w * a1[None, :]).reshape(FIELD, EMB, 12)
    b1f = lin1_b * a1 + bn1_b
    a2 = bn2_g / jnp.sqrt(1.0 + EPS)
    w2f = lin2_w * a2[None, :]
    b2f = lin2_b * a2 + bn2_b
    eye8 = jnp.eye(8, dtype=jnp.float32)
    w1k = jnp.einsum('ab,fej->faebj', eye8, w1f).reshape(FIELD, 128, 96)
    b1d = jnp.tile(b1f, 8)
    w2d = jnp.einsum('ab,ej->aebj', eye8, w2f).reshape(96, 64)
    b2d = jnp.tile(b2f, 8)
    d16 = jnp.einsum('ab,e->aeb', eye8, jnp.ones(EMB)).reshape(128, 8)
    d8 = jnp.einsum('ab,e->aeb', eye8, jnp.ones(8)).reshape(64, 8)

    fd, fs = _tc_dense(emb3d, first3d, w1k, b1d, w2d, b2d, d16, d8)
    tot = fd.reshape(BATCH) + fs.reshape(BATCH) + bias[0]
    return jax.nn.sigmoid(tot)
